# Initial kernel scaffold; baseline (speedup 1.0000x reference)
import jax
import jax.numpy as jnp
from jax.experimental import pallas as pl


def kernel(x, edge_index, batch, params):
    raise NotImplementedError("write your pallas kernel here")


# speedup vs baseline: 6.9115x; 6.9115x over previous
import functools

import jax
import jax.numpy as jnp
from jax import lax
from jax.experimental import pallas as pl
from jax.experimental.pallas import tpu as pltpu
from jax.experimental.pallas import tpu_sc as plsc

N = 10000
E = 320000
F_IN = 128
DIM = 16
NCLS = 10

NC = 2
NS = 16
NW = NC * NS
EW = E // NW
CH = 80
NCH = EW // CH

_mesh = lambda: plsc.VectorSubcoreMesh(core_axis_name="c", subcore_axis_name="s")
_SC_PARAMS = pltpu.CompilerParams(use_tc_tiling_on_sc=False)


def _segsum_sc(z, src3, dst3, zeros):
  @functools.partial(
      pl.kernel,
      mesh=_mesh(),
      compiler_params=_SC_PARAMS,
      out_type=jax.ShapeDtypeStruct((NC, N, DIM), jnp.float32),
      scratch_types=[
          pltpu.VMEM((NCH, CH), jnp.int32),
          pltpu.VMEM((NCH, CH), jnp.int32),
          pltpu.VMEM((CH, DIM), jnp.float32),
          pltpu.VMEM_SHARED((N, DIM), jnp.float32),
          pltpu.SemaphoreType.DMA,
      ],
  )
  def k(z_hbm, src_hbm, dst_hbm, zeros_hbm, out_hbm, srcv, dstv, rows, acc, sem):
    c = lax.axis_index("c")
    s = lax.axis_index("s")
    wid = c * NS + s
    rpw = N // NS
    pltpu.sync_copy(zeros_hbm.at[pl.ds(s * rpw, rpw)], acc.at[pl.ds(s * rpw, rpw)])
    pltpu.sync_copy(src_hbm.at[wid], srcv)
    pltpu.sync_copy(dst_hbm.at[wid], dstv)
    plsc.subcore_barrier()

    def body(j, carry):
      pltpu.async_copy(z_hbm.at[srcv.at[j]], rows, sem).wait()
      pltpu.sync_copy(rows, acc.at[dstv.at[j]], add=True)
      return carry

    lax.fori_loop(0, NCH, body, 0, unroll=False)
    plsc.subcore_barrier()
    pltpu.sync_copy(acc.at[pl.ds(s * rpw, rpw)], out_hbm.at[c].at[pl.ds(s * rpw, rpw)])

  return k(z, src3, dst3, zeros)


def _edge_gather_sc(h, src3, dst3):
  @functools.partial(
      pl.kernel,
      mesh=_mesh(),
      compiler_params=_SC_PARAMS,
      out_type=jax.ShapeDtypeStruct((E, 2 * DIM), jnp.float32),
      scratch_types=[
          pltpu.VMEM((NCH, CH), jnp.int32),
          pltpu.VMEM((NCH, CH), jnp.int32),
          pltpu.VMEM((CH, DIM), jnp.float32),
          pltpu.VMEM((CH, DIM), jnp.float32),
          pltpu.SemaphoreType.DMA,
          pltpu.SemaphoreType.DMA,
      ],
  )
  def k(h_hbm, src_hbm, dst_hbm, e_hbm, srcv, dstv, bufa, bufb, sema, semb):
    c = lax.axis_index("c")
    s = lax.axis_index("s")
    wid = c * NS + s
    pltpu.sync_copy(src_hbm.at[wid], srcv)
    pltpu.sync_copy(dst_hbm.at[wid], dstv)

    def body(j, carry):
      base = wid * EW + j * CH
      cpa = pltpu.async_copy(h_hbm.at[srcv.at[j]], bufa, sema)
      cpb = pltpu.async_copy(h_hbm.at[dstv.at[j]], bufb, semb)
      cpa.wait()
      pltpu.sync_copy(bufa, e_hbm.at[pl.ds(base, CH), pl.ds(0, DIM)])
      cpb.wait()
      pltpu.sync_copy(bufb, e_hbm.at[pl.ds(base, CH), pl.ds(DIM, DIM)])
      return carry

    lax.fori_loop(0, NCH, body, 0, unroll=False)

  return k(h, src3, dst3)


_BLK = 2000


def _proj_tc(x, v1):
  def body(x_ref, v_ref, o_ref):
    o_ref[...] = jnp.dot(x_ref[...], v_ref[...], preferred_element_type=jnp.float32)

  return pl.pallas_call(
      body,
      grid=(N // _BLK,),
      in_specs=[
          pl.BlockSpec((_BLK, F_IN), lambda i: (i, 0)),
          pl.BlockSpec((F_IN, DIM), lambda i: (0, 0)),
      ],
      out_specs=pl.BlockSpec((_BLK, DIM), lambda i: (i, 0)),
      out_shape=jax.ShapeDtypeStruct((N, DIM), jnp.float32),
  )(x, v1)


def _layer_tc(z, agg, cvec, w2, b2, vnext):
  def body(z_ref, a_ref, c_ref, w2_ref, b2_ref, vn_ref, o_ref):
    a = z_ref[...] + a_ref[0] + a_ref[1] + c_ref[...]
    h = jnp.maximum(a, 0.0)
    h2 = jnp.dot(h, w2_ref[...], preferred_element_type=jnp.float32) + b2_ref[...]
    h2 = jnp.maximum(h2, 0.0)
    o_ref[...] = jnp.dot(h2, vn_ref[...], preferred_element_type=jnp.float32)

  return pl.pallas_call(
      body,
      grid=(N // _BLK,),
      in_specs=[
          pl.BlockSpec((_BLK, DIM), lambda i: (i, 0)),
          pl.BlockSpec((NC, _BLK, DIM), lambda i: (0, i, 0)),
          pl.BlockSpec((1, DIM), lambda i: (0, 0)),
          pl.BlockSpec((DIM, DIM), lambda i: (0, 0)),
          pl.BlockSpec((1, DIM), lambda i: (0, 0)),
          pl.BlockSpec((DIM, DIM), lambda i: (0, 0)),
      ],
      out_specs=pl.BlockSpec((_BLK, DIM), lambda i: (i, 0)),
      out_shape=jax.ShapeDtypeStruct((N, DIM), jnp.float32),
  )(z, agg, cvec.reshape(1, DIM), w2, b2.reshape(1, DIM), vnext)


def _last_tc(z, agg, cvec, w2, b2, l1w, l1b):
  def body(z_ref, a_ref, c_ref, w2_ref, b2_ref, lw_ref, lb_ref, o_ref):
    a = z_ref[...] + a_ref[0] + a_ref[1] + c_ref[...]
    h = jnp.maximum(a, 0.0)
    h2 = jnp.dot(h, w2_ref[...], preferred_element_type=jnp.float32) + b2_ref[...]
    h2 = jnp.maximum(h2, 0.0)
    h3 = jnp.dot(h2, lw_ref[...], preferred_element_type=jnp.float32) + lb_ref[...]
    o_ref[...] = jnp.maximum(h3, 0.0)

  return pl.pallas_call(
      body,
      grid=(N // _BLK,),
      in_specs=[
          pl.BlockSpec((_BLK, DIM), lambda i: (i, 0)),
          pl.BlockSpec((NC, _BLK, DIM), lambda i: (0, i, 0)),
          pl.BlockSpec((1, DIM), lambda i: (0, 0)),
          pl.BlockSpec((DIM, DIM), lambda i: (0, 0)),
          pl.BlockSpec((1, DIM), lambda i: (0, 0)),
          pl.BlockSpec((DIM, DIM), lambda i: (0, 0)),
          pl.BlockSpec((1, DIM), lambda i: (0, 0)),
      ],
      out_specs=pl.BlockSpec((_BLK, DIM), lambda i: (i, 0)),
      out_shape=jax.ShapeDtypeStruct((N, DIM), jnp.float32),
  )(z, agg, cvec.reshape(1, DIM), w2, b2.reshape(1, DIM), l1w, l1b.reshape(1, DIM))


_EBLK = 4000


def _out_tc(e, l2w, l2b):
  def body(e_ref, w_ref, b_ref, o_ref):
    o_ref[...] = (
        jnp.dot(e_ref[...], w_ref[...], preferred_element_type=jnp.float32)
        + b_ref[...]
    )

  return pl.pallas_call(
      body,
      grid=(E // _EBLK,),
      in_specs=[
          pl.BlockSpec((_EBLK, 2 * DIM), lambda i: (i, 0)),
          pl.BlockSpec((2 * DIM, NCLS), lambda i: (0, 0)),
          pl.BlockSpec((1, NCLS), lambda i: (0, 0)),
      ],
      out_specs=pl.BlockSpec((_EBLK, NCLS), lambda i: (i, 0)),
      out_shape=jax.ShapeDtypeStruct((E, NCLS), jnp.float32),
  )(e, l2w, l2b.reshape(1, NCLS))


def _fold(p):
  s = p["g"] * lax.rsqrt(p["v"] + 1e-5)
  return p["w1"] * s[None, :], p["b1"] * s + p["be"] - p["m"] * s


def kernel(x, edge_index, batch, params):
  del batch
  src3 = edge_index[0].reshape(NW, NCH, CH)
  dst3 = edge_index[1].reshape(NW, NCH, CH)
  zeros = jnp.zeros((N, DIM), jnp.float32)

  convs = [params["c1"], params["c2"], params["c3"], params["c4"]]
  folded = [_fold(p) for p in convs]

  z = _proj_tc(x, folded[0][0])
  for i in range(3):
    agg = _segsum_sc(z, src3, dst3, zeros)
    z = _layer_tc(z, agg, folded[i][1], convs[i]["w2"], convs[i]["b2"],
                  folded[i + 1][0])
  agg = _segsum_sc(z, src3, dst3, zeros)
  l1w, l1b = params["l1"]
  h5 = _last_tc(z, agg, folded[3][1], convs[3]["w2"], convs[3]["b2"], l1w, l1b)

  e = _edge_gather_sc(h5, src3, dst3)
  l2w, l2b = params["l2"]
  out = _out_tc(e, l2w, l2b)
  return (out, e)


# speedup vs baseline: 11.7702x; 1.7030x over previous
import functools

import jax
import jax.numpy as jnp
from jax import lax
from jax.experimental import pallas as pl
from jax.experimental.pallas import tpu as pltpu
from jax.experimental.pallas import tpu_sc as plsc

N = 10000
E = 320000
F_IN = 128
DIM = 16
NCLS = 10

NC = 2
NS = 16
NW = NC * NS
EW = E // NW
CH = 80
NCH = EW // CH
GRP = 5

_mesh = lambda: plsc.VectorSubcoreMesh(core_axis_name="c", subcore_axis_name="s")
_SC_PARAMS = pltpu.CompilerParams(use_tc_tiling_on_sc=False)


def _segsum_sc(z, src3, dst3, zeros):
  @functools.partial(
      pl.kernel,
      mesh=_mesh(),
      compiler_params=_SC_PARAMS,
      out_type=jax.ShapeDtypeStruct((NC, N, DIM), jnp.float32),
      scratch_types=[
          pltpu.VMEM((NCH, CH), jnp.int32),
          pltpu.VMEM((NCH, CH), jnp.int32),
          pltpu.VMEM((GRP * CH, DIM), jnp.float32),
          pltpu.VMEM_SHARED((N, DIM), jnp.float32),
          pltpu.VMEM_SHARED((N, DIM), jnp.float32),
          pltpu.SemaphoreType.DMA((GRP,)),
          pltpu.SemaphoreType.DMA((GRP,)),
      ],
  )
  def k(z_hbm, src_hbm, dst_hbm, zeros_hbm, out_hbm, srcv, dstv, rows, acc,
        ztab, gsem, ssem):
    c = lax.axis_index("c")
    s = lax.axis_index("s")
    wid = c * NS + s
    rpw = N // NS
    pltpu.sync_copy(zeros_hbm.at[pl.ds(s * rpw, rpw)], acc.at[pl.ds(s * rpw, rpw)])
    pltpu.sync_copy(z_hbm.at[pl.ds(s * rpw, rpw)], ztab.at[pl.ds(s * rpw, rpw)])
    pltpu.sync_copy(src_hbm.at[wid], srcv)
    pltpu.sync_copy(dst_hbm.at[wid], dstv)
    plsc.subcore_barrier()

    def body(g, carry):
      gcps = [
          pltpu.async_copy(ztab.at[srcv.at[g * GRP + b]],
                           rows.at[pl.ds(b * CH, CH)], gsem.at[b])
          for b in range(GRP)
      ]
      scps = []
      for b in range(GRP):
        gcps[b].wait()
        scps.append(
            pltpu.async_copy(rows.at[pl.ds(b * CH, CH)],
                             acc.at[dstv.at[g * GRP + b]], ssem.at[b],
                             add=True))
      for cp in scps:
        cp.wait()
      return carry

    lax.fori_loop(0, NCH // GRP, body, 0, unroll=False)
    plsc.subcore_barrier()
    pltpu.sync_copy(acc.at[pl.ds(s * rpw, rpw)], out_hbm.at[c].at[pl.ds(s * rpw, rpw)])

  return k(z, src3, dst3, zeros)


def _edge_gather_sc(h, src3, dst3):
  @functools.partial(
      pl.kernel,
      mesh=_mesh(),
      compiler_params=_SC_PARAMS,
      out_type=jax.ShapeDtypeStruct((E, 2 * DIM), jnp.float32),
      scratch_types=[
          pltpu.VMEM((NCH, CH), jnp.int32),
          pltpu.VMEM((NCH, CH), jnp.int32),
          pltpu.VMEM((GRP * CH, DIM), jnp.float32),
          pltpu.VMEM((GRP * CH, DIM), jnp.float32),
          pltpu.VMEM_SHARED((N, DIM), jnp.float32),
          pltpu.SemaphoreType.DMA((GRP,)),
          pltpu.SemaphoreType.DMA((GRP,)),
          pltpu.SemaphoreType.DMA((GRP,)),
          pltpu.SemaphoreType.DMA((GRP,)),
      ],
  )
  def k(h_hbm, src_hbm, dst_hbm, e_hbm, srcv, dstv, bufa, bufb, htab, sema,
        semb, wsema, wsemb):
    c = lax.axis_index("c")
    s = lax.axis_index("s")
    wid = c * NS + s
    rpw = N // NS
    pltpu.sync_copy(h_hbm.at[pl.ds(s * rpw, rpw)], htab.at[pl.ds(s * rpw, rpw)])
    pltpu.sync_copy(src_hbm.at[wid], srcv)
    pltpu.sync_copy(dst_hbm.at[wid], dstv)
    plsc.subcore_barrier()

    def body(g, carry):
      gcps = []
      for b in range(GRP):
        j = g * GRP + b
        sl = pl.ds(b * CH, CH)
        gcps.append((pltpu.async_copy(htab.at[srcv.at[j]], bufa.at[sl], sema.at[b]),
                     pltpu.async_copy(htab.at[dstv.at[j]], bufb.at[sl], semb.at[b])))
      wcps = []
      for b in range(GRP):
        base = wid * EW + (g * GRP + b) * CH
        sl = pl.ds(b * CH, CH)
        cpa, cpb = gcps[b]
        cpa.wait()
        wcps.append(pltpu.async_copy(
            bufa.at[sl], e_hbm.at[pl.ds(base, CH), pl.ds(0, DIM)], wsema.at[b]))
        cpb.wait()
        wcps.append(pltpu.async_copy(
            bufb.at[sl], e_hbm.at[pl.ds(base, CH), pl.ds(DIM, DIM)], wsemb.at[b]))
      for cp in wcps:
        cp.wait()
      return carry

    lax.fori_loop(0, NCH // GRP, body, 0, unroll=False)

  return k(h, src3, dst3)


_BLK = 2000


def _proj_tc(x, v1):
  def body(x_ref, v_ref, o_ref):
    o_ref[...] = jnp.dot(x_ref[...], v_ref[...], preferred_element_type=jnp.float32)

  return pl.pallas_call(
      body,
      grid=(N // _BLK,),
      in_specs=[
          pl.BlockSpec((_BLK, F_IN), lambda i: (i, 0)),
          pl.BlockSpec((F_IN, DIM), lambda i: (0, 0)),
      ],
      out_specs=pl.BlockSpec((_BLK, DIM), lambda i: (i, 0)),
      out_shape=jax.ShapeDtypeStruct((N, DIM), jnp.float32),
  )(x, v1)


def _layer_tc(z, agg, cvec, w2, b2, vnext):
  def body(z_ref, a_ref, c_ref, w2_ref, b2_ref, vn_ref, o_ref):
    a = z_ref[...] + a_ref[0] + a_ref[1] + c_ref[...]
    h = jnp.maximum(a, 0.0)
    h2 = jnp.dot(h, w2_ref[...], preferred_element_type=jnp.float32) + b2_ref[...]
    h2 = jnp.maximum(h2, 0.0)
    o_ref[...] = jnp.dot(h2, vn_ref[...], preferred_element_type=jnp.float32)

  return pl.pallas_call(
      body,
      grid=(N // _BLK,),
      in_specs=[
          pl.BlockSpec((_BLK, DIM), lambda i: (i, 0)),
          pl.BlockSpec((NC, _BLK, DIM), lambda i: (0, i, 0)),
          pl.BlockSpec((1, DIM), lambda i: (0, 0)),
          pl.BlockSpec((DIM, DIM), lambda i: (0, 0)),
          pl.BlockSpec((1, DIM), lambda i: (0, 0)),
          pl.BlockSpec((DIM, DIM), lambda i: (0, 0)),
      ],
      out_specs=pl.BlockSpec((_BLK, DIM), lambda i: (i, 0)),
      out_shape=jax.ShapeDtypeStruct((N, DIM), jnp.float32),
  )(z, agg, cvec.reshape(1, DIM), w2, b2.reshape(1, DIM), vnext)


def _last_tc(z, agg, cvec, w2, b2, l1w, l1b):
  def body(z_ref, a_ref, c_ref, w2_ref, b2_ref, lw_ref, lb_ref, o_ref):
    a = z_ref[...] + a_ref[0] + a_ref[1] + c_ref[...]
    h = jnp.maximum(a, 0.0)
    h2 = jnp.dot(h, w2_ref[...], preferred_element_type=jnp.float32) + b2_ref[...]
    h2 = jnp.maximum(h2, 0.0)
    h3 = jnp.dot(h2, lw_ref[...], preferred_element_type=jnp.float32) + lb_ref[...]
    o_ref[...] = jnp.maximum(h3, 0.0)

  return pl.pallas_call(
      body,
      grid=(N // _BLK,),
      in_specs=[
          pl.BlockSpec((_BLK, DIM), lambda i: (i, 0)),
          pl.BlockSpec((NC, _BLK, DIM), lambda i: (0, i, 0)),
          pl.BlockSpec((1, DIM), lambda i: (0, 0)),
          pl.BlockSpec((DIM, DIM), lambda i: (0, 0)),
          pl.BlockSpec((1, DIM), lambda i: (0, 0)),
          pl.BlockSpec((DIM, DIM), lambda i: (0, 0)),
          pl.BlockSpec((1, DIM), lambda i: (0, 0)),
      ],
      out_specs=pl.BlockSpec((_BLK, DIM), lambda i: (i, 0)),
      out_shape=jax.ShapeDtypeStruct((N, DIM), jnp.float32),
  )(z, agg, cvec.reshape(1, DIM), w2, b2.reshape(1, DIM), l1w, l1b.reshape(1, DIM))


_EBLK = 4000


def _out_tc(e, l2w, l2b):
  def body(e_ref, w_ref, b_ref, o_ref):
    o_ref[...] = (
        jnp.dot(e_ref[...], w_ref[...], preferred_element_type=jnp.float32)
        + b_ref[...]
    )

  return pl.pallas_call(
      body,
      grid=(E // _EBLK,),
      in_specs=[
          pl.BlockSpec((_EBLK, 2 * DIM), lambda i: (i, 0)),
          pl.BlockSpec((2 * DIM, NCLS), lambda i: (0, 0)),
          pl.BlockSpec((1, NCLS), lambda i: (0, 0)),
      ],
      out_specs=pl.BlockSpec((_EBLK, NCLS), lambda i: (i, 0)),
      out_shape=jax.ShapeDtypeStruct((E, NCLS), jnp.float32),
  )(e, l2w, l2b.reshape(1, NCLS))


def _fold(p):
  s = p["g"] * lax.rsqrt(p["v"] + 1e-5)
  return p["w1"] * s[None, :], p["b1"] * s + p["be"] - p["m"] * s


def kernel(x, edge_index, batch, params):
  del batch
  src3 = edge_index[0].reshape(NW, NCH, CH)
  dst3 = edge_index[1].reshape(NW, NCH, CH)
  zeros = jnp.zeros((N, DIM), jnp.float32)

  convs = [params["c1"], params["c2"], params["c3"], params["c4"]]
  folded = [_fold(p) for p in convs]

  z = _proj_tc(x, folded[0][0])
  for i in range(3):
    agg = _segsum_sc(z, src3, dst3, zeros)
    z = _layer_tc(z, agg, folded[i][1], convs[i]["w2"], convs[i]["b2"],
                  folded[i + 1][0])
  agg = _segsum_sc(z, src3, dst3, zeros)
  l1w, l1b = params["l1"]
  h5 = _last_tc(z, agg, folded[3][1], convs[3]["w2"], convs[3]["b2"], l1w, l1b)

  e = _edge_gather_sc(h5, src3, dst3)
  l2w, l2b = params["l2"]
  out = _out_tc(e, l2w, l2b)
  return (out, e)
